# baseline (speedup 1.0000x reference)
import functools

import jax
import jax.numpy as jnp
from jax import lax
from jax.experimental import pallas as pl
from jax.experimental.pallas import tpu as pltpu
from jax.experimental.pallas import tpu_sc as plsc

B = 4096
D = 32
NREL = 64
DD = D * D
NW = 32
BPW = B // NW
RBLK = 512
NBLK = B // RBLK


def _sc_gather_body(users_h, pos_h, neg_h, heads_h, pt_h, nt_h,
                    ue_h, ie_h, ke_h,
                    o_u, o_pi, o_pik, o_ni, o_nik, o_h, o_pt, o_nt,
                    iu, ip, ineg, ih, ipt, int_,
                    r0, r1, r2, r3, r4, r5, r6, r7, sem):
    c = lax.axis_index("c")
    s = lax.axis_index("s")
    wid = s * 2 + c
    base = wid * BPW
    for hb, vb in ((users_h, iu), (pos_h, ip), (neg_h, ineg),
                   (heads_h, ih), (pt_h, ipt), (nt_h, int_)):
        pltpu.sync_copy(hb.at[pl.ds(base, BPW)], vb)
    gathers = ((ue_h, iu, r0), (ie_h, ip, r1), (ke_h, ip, r2),
               (ie_h, ineg, r3), (ke_h, ineg, r4),
               (ke_h, ih, r5), (ke_h, ipt, r6), (ke_h, int_, r7))
    copies = [pltpu.async_copy(tbl.at[vb], rv, sem) for tbl, vb, rv in gathers]
    for cp in copies:
        cp.wait()
    for rv, oh in zip((r0, r1, r2, r3, r4, r5, r6, r7),
                      (o_u, o_pi, o_pik, o_ni, o_nik, o_h, o_pt, o_nt)):
        pltpu.sync_copy(rv, oh.at[pl.ds(base, BPW)])


def _sc_gather(users, pos_items, neg_items, heads, pos_tails, neg_tails,
               user_embed, item_embed, kg_entity_embed):
    mesh = plsc.VectorSubcoreMesh(core_axis_name="c", subcore_axis_name="s")
    f = pl.kernel(
        _sc_gather_body,
        out_type=[jax.ShapeDtypeStruct((B, D), jnp.float32)] * 8,
        mesh=mesh,
        scratch_types=(
            [pltpu.VMEM((BPW,), jnp.int32)] * 6
            + [pltpu.VMEM((BPW, D), jnp.float32)] * 8
            + [pltpu.SemaphoreType.DMA]
        ),
        compiler_params=pltpu.CompilerParams(use_tc_tiling_on_sc=False),
    )
    return f(users, pos_items, neg_items, heads, pos_tails, neg_tails,
             user_embed, item_embed, kg_entity_embed)


def _l2n(x):
    n = jnp.sqrt(jnp.sum(x * x, axis=1, keepdims=True))
    return x / jnp.maximum(n, 1e-12)


def _rowwork_body(rel_ref, pie_ref, pik_ref, nie_ref, nik_ref,
                  h_ref, pt_ref, nt_ref, rel_emb_ref, wflat_ref,
                  picomb_ref, nicomb_ref, he_ref, re_ref, pte_ref, nte_ref):
    rel = rel_ref[0, 0, :]
    onehot = (rel[:, None] == lax.broadcasted_iota(jnp.int32, (RBLK, NREL), 1)
              ).astype(jnp.float32)
    re_ref[...] = _l2n(jnp.dot(onehot, rel_emb_ref[...],
                               preferred_element_type=jnp.float32))
    wg = jnp.dot(onehot, wflat_ref[...], preferred_element_type=jnp.float32)
    R = (lax.broadcasted_iota(jnp.int32, (D, DD), 1) % D
         == lax.broadcasted_iota(jnp.int32, (D, DD), 0)).astype(jnp.float32)
    S = (lax.broadcasted_iota(jnp.int32, (DD, D), 0) // D
         == lax.broadcasted_iota(jnp.int32, (DD, D), 1)).astype(jnp.float32)
    for x_ref, o_ref in ((h_ref, he_ref), (pt_ref, pte_ref), (nt_ref, nte_ref)):
        xt = jnp.dot(x_ref[...], R, preferred_element_type=jnp.float32)
        proj = jnp.dot(xt * wg, S, preferred_element_type=jnp.float32)
        o_ref[...] = _l2n(proj)
    picomb_ref[...] = pie_ref[...] + pik_ref[...]
    nicomb_ref[...] = nie_ref[...] + nik_ref[...]


def _matmul_body(u_ref, c_ref, o_ref):
    o_ref[...] = lax.dot_general(u_ref[...], c_ref[...],
                                 (((1,), (1,)), ((), ())),
                                 preferred_element_type=jnp.float32)


def kernel(users, pos_items, neg_items, heads, relations, pos_tails, neg_tails,
           user_embed, item_embed, kg_entity_embed, kg_relation_embed, trans_W):
    u_e, pie, pik, nie, nik, h_raw, pt_raw, nt_raw = _sc_gather(
        users, pos_items, neg_items, heads, pos_tails, neg_tails,
        user_embed, item_embed, kg_entity_embed)
    wflat = trans_W.transpose(0, 2, 1).reshape(NREL, DD)
    rel3 = relations.reshape(NBLK, 1, RBLK)
    row_spec = pl.BlockSpec((RBLK, D), lambda i: (i, 0))
    picomb, nicomb, h_e, r_e, pt_e, nt_e = pl.pallas_call(
        _rowwork_body,
        grid=(NBLK,),
        in_specs=[pl.BlockSpec((1, 1, RBLK), lambda i: (i, 0, 0))]
        + [row_spec] * 7
        + [pl.BlockSpec((NREL, D), lambda i: (0, 0)),
           pl.BlockSpec((NREL, DD), lambda i: (0, 0))],
        out_specs=[row_spec] * 6,
        out_shape=[jax.ShapeDtypeStruct((B, D), jnp.float32)] * 6,
    )(rel3, pie, pik, nie, nik, h_raw, pt_raw, nt_raw,
      kg_relation_embed, wflat)
    preds = pl.pallas_call(
        _matmul_body,
        grid=(NBLK,),
        in_specs=[pl.BlockSpec((RBLK, D), lambda i: (i, 0)),
                  pl.BlockSpec((B, D), lambda i: (0, 0))],
        out_specs=pl.BlockSpec((RBLK, B), lambda i: (i, 0)),
        out_shape=jax.ShapeDtypeStruct((B, B), jnp.float32),
    )(u_e, picomb)
    return (u_e, picomb, nicomb, h_e, r_e, pt_e, nt_e, preds)
